# baseline (speedup 1.0000x reference)
import math

import jax
import jax.numpy as jnp
from jax import lax
from jax.experimental import pallas as pl
from jax.experimental.pallas import tpu as pltpu
from jax.experimental.pallas import tpu_sc as plsc

V = 1000000
D = 64
L = 16
NC, NS = 2, 16
NW = NC * NS
R, S = 16384, 50
RPW = R // NW
P = 4
CH = P * S
NCHUNK = RPW // P
SCALE = math.sqrt(D)

NBLK = V // 128
BPT = (NBLK + NW - 1) // NW
VTAIL = NBLK * 128


def _widen_body(tt_hbm, wide_hbm, rb0, rb1, wb, rbt, sem0, sem1):
    wid = lax.axis_index("s") * NC + lax.axis_index("c")
    rb = (rb0, rb1)
    sems = (sem0, sem1)

    def v0_of(k):
        return pl.multiple_of((wid + k * NW) * 128, 128)

    for b in range(2):
        @pl.when(wid + b * NW < NBLK)
        def _pre():
            pltpu.async_copy(
                tt_hbm.at[:, pl.ds(v0_of(b), 128)], rb[b], sems[b])

    dcols = [jax.lax.iota(jnp.int32, L) + j * L for j in range(D // L)]

    @pl.loop(0, BPT, step=2)
    def _blocks(k):
        for b in range(2):
            kb = k + b

            @pl.when(wid + kb * NW < NBLK)
            def _do():
                pltpu.make_async_copy(
                    tt_hbm.at[:, pl.ds(v0_of(kb), 128)], rb[b], sems[b]).wait()

                @plsc.parallel_loop(0, 128, 1)
                def _row(v):
                    vv = jnp.full((L,), 0, jnp.int32) + v
                    for j in range(D // L):
                        wb[v, pl.ds(j * L, L)] = plsc.load_gather(
                            rb[b], [dcols[j], vv])

                pltpu.sync_copy(wb, wide_hbm.at[pl.ds(v0_of(kb), 128)])

                @pl.when(wid + (kb + 2) * NW < NBLK)
                def _fire():
                    pltpu.async_copy(
                        tt_hbm.at[:, pl.ds(v0_of(kb + 2), 128)],
                        rb[b], sems[b])

    @pl.when(wid == 0)
    def _tail():
        pltpu.sync_copy(tt_hbm.at[:, pl.ds(VTAIL, V - VTAIL)], rbt)

        @plsc.parallel_loop(0, V - VTAIL, 1)
        def _trow(v):
            vv = jnp.full((L,), 0, jnp.int32) + v
            for j in range(D // L):
                wb[v, pl.ds(j * L, L)] = plsc.load_gather(rbt, [dcols[j], vv])

        pltpu.sync_copy(wb.at[pl.ds(0, V - VTAIL)],
                        wide_hbm.at[pl.ds(VTAIL, V - VTAIL)])


def _lookup_body(table_hbm, idx_hbm, out_hbm,
                 idx0, idx1, rows0, rows1, blk, sem0, sem1):
    wid = lax.axis_index("s") * NC + lax.axis_index("c")
    rbase = wid * RPW
    fbase = rbase * S
    idx = (idx0, idx1)
    rows = (rows0, rows1)
    sems = (sem0, sem1)

    for b in range(2):
        pltpu.sync_copy(idx_hbm.at[pl.ds(fbase + b * CH, CH)], idx[b])
        pltpu.async_copy(table_hbm.at[idx[b]], rows[b], sems[b])

    @pl.loop(0, NCHUNK, step=2)
    def _chunks(g):
        for b in range(2):
            gb = g + b
            pltpu.make_async_copy(
                table_hbm.at[idx[b]], rows[b], sems[b]).wait()

            for p in range(P):
                @plsc.parallel_loop(0, S, 1, unroll=4)
                def _scale_tok(s):
                    r = p * S + s
                    for j in range(D // L):
                        blk[p, s, pl.ds(j * L, L)] = (
                            rows[b][r, pl.ds(j * L, L)] * SCALE)

            pltpu.sync_copy(blk, out_hbm.at[pl.ds(rbase + gb * P, P)])

            @pl.when(gb + 2 < NCHUNK)
            def _fire():
                nxt = fbase + (gb + 2) * CH
                pltpu.sync_copy(idx_hbm.at[pl.ds(nxt, CH)], idx[b])
                pltpu.async_copy(table_hbm.at[idx[b]], rows[b], sems[b])


def kernel(tokens, table):
    tok_flat = tokens.reshape(-1)
    mesh = plsc.VectorSubcoreMesh(core_axis_name="c", subcore_axis_name="s")

    widen = pl.kernel(
        _widen_body,
        out_type=jax.ShapeDtypeStruct((V, 128), jnp.float32),
        mesh=mesh,
        scratch_types=[
            pltpu.VMEM((D, 128), jnp.float32),
            pltpu.VMEM((D, 128), jnp.float32),
            pltpu.VMEM((128, 128), jnp.float32),
            pltpu.VMEM((D, V - VTAIL), jnp.float32),
            pltpu.SemaphoreType.DMA,
            pltpu.SemaphoreType.DMA,
        ],
        compiler_params=pltpu.CompilerParams(needs_layout_passes=False),
    )
    wide = widen(table.T)

    lookup = pl.kernel(
        _lookup_body,
        out_type=jax.ShapeDtypeStruct((R, S, D), jnp.float32),
        mesh=mesh,
        scratch_types=[
            pltpu.VMEM((CH,), jnp.int32),
            pltpu.VMEM((CH,), jnp.int32),
            pltpu.VMEM((CH, 128), jnp.float32),
            pltpu.VMEM((CH, 128), jnp.float32),
            pltpu.VMEM((P, S, D), jnp.float32),
            pltpu.SemaphoreType.DMA,
            pltpu.SemaphoreType.DMA,
        ],
    )
    return lookup(wide, tok_flat)


# speedup vs baseline: 1.3609x; 1.3609x over previous
import math

import jax
import jax.numpy as jnp
from jax import lax
from jax.experimental import pallas as pl
from jax.experimental.pallas import tpu as pltpu
from jax.experimental.pallas import tpu_sc as plsc

D = 64
L = 16
NC, NS = 2, 16
NW = NC * NS
R, S = 16384, 50
RPW = R // NW
P = 4
CH = P * S
NCHUNK = RPW // P
SCALE = math.sqrt(D)


def _emb_body(table_hbm, idx_hbm, out_hbm,
              idx0, idx1, rows0, rows1, blk, sem0, sem1):
    wid = lax.axis_index("s") * NC + lax.axis_index("c")
    rbase = wid * RPW
    fbase = rbase * S
    idx = (idx0, idx1)
    rows = (rows0, rows1)
    sems = (sem0, sem1)

    for b in range(2):
        pltpu.sync_copy(idx_hbm.at[pl.ds(fbase + b * CH, CH)], idx[b])
        pltpu.async_copy(table_hbm.at[idx[b]], rows[b], sems[b])

    @pl.loop(0, NCHUNK, step=2)
    def _chunks(g):
        for b in range(2):
            gb = g + b
            pltpu.make_async_copy(
                table_hbm.at[idx[b]], rows[b], sems[b]).wait()

            for p in range(P):
                @plsc.parallel_loop(0, S, 1, unroll=4)
                def _scale_tok(s):
                    r = p * S + s
                    for j in range(D // L):
                        blk[p, s, pl.ds(j * L, L)] = (
                            rows[b][r, pl.ds(j * L, L)] * SCALE)

            pltpu.sync_copy(blk, out_hbm.at[pl.ds(rbase + gb * P, P)])

            @pl.when(gb + 2 < NCHUNK)
            def _fire():
                nxt = fbase + (gb + 2) * CH
                pltpu.sync_copy(idx_hbm.at[pl.ds(nxt, CH)], idx[b])
                pltpu.async_copy(table_hbm.at[idx[b]], rows[b], sems[b])


def kernel(tokens, table):
    tok_flat = tokens.reshape(-1)
    wide = jnp.pad(table, ((0, 0), (0, 64)))
    mesh = plsc.VectorSubcoreMesh(core_axis_name="c", subcore_axis_name="s")
    k = pl.kernel(
        _emb_body,
        out_type=jax.ShapeDtypeStruct((R, S, D), jnp.float32),
        mesh=mesh,
        scratch_types=[
            pltpu.VMEM((CH,), jnp.int32),
            pltpu.VMEM((CH,), jnp.int32),
            pltpu.VMEM((CH, 128), jnp.float32),
            pltpu.VMEM((CH, 128), jnp.float32),
            pltpu.VMEM((P, S, D), jnp.float32),
            pltpu.SemaphoreType.DMA,
            pltpu.SemaphoreType.DMA,
        ],
    )
    return k(wide, tok_flat)
